# baseline (speedup 1.0000x reference)
import jax
import jax.numpy as jnp
from jax.experimental import pallas as pl
from jax.experimental.pallas import tpu as pltpu

_EPS = 1e-6
_D = 256
_BM = 512


def _mcl_kernel(f_ref, c_ref, l_ref, out_ref, cs_ref, cn_ref, acc_s, acc_n):
    i = pl.program_id(0)
    nsteps = pl.num_programs(0)

    @pl.when(i == 0)
    def _init():
        acc_s[0, 0] = 0.0
        acc_n[0, 0] = 0.0
        c = c_ref[...]
        cs_ref[...] = -2.0 * c
        cn_ref[...] = (
            jnp.sum(c * c, axis=1) - (2.0 * _EPS) * jnp.sum(c, axis=1)
        )[None, :]

    f = f_ref[...]
    dot = jax.lax.dot_general(
        f, cs_ref[...], (((1,), (1,)), ((), ())),
        preferred_element_type=jnp.float32,
    )
    t = dot + cn_ref[...]
    m = jnp.min(t, axis=1, keepdims=True)
    rn = jnp.sum(f * f, axis=1, keepdims=True) + (2.0 * _EPS) * jnp.sum(
        f, axis=1, keepdims=True
    )
    min_d = jnp.sqrt(jnp.maximum(m + rn + (_D * _EPS * _EPS), 0.0))
    mask = (l_ref[...] == 0).astype(jnp.float32)
    acc_s[0, 0] += jnp.sum(mask * min_d)
    acc_n[0, 0] += jnp.sum(mask)

    @pl.when(i == nsteps - 1)
    def _fin():
        out_ref[0, 0] = acc_s[0, 0] / (acc_n[0, 0] + 1e-5)


def kernel(features, labels, centers):
    batch, d = features.shape
    ncenters = centers.shape[0]
    labels2 = labels.reshape(batch, 1)
    nsteps = batch // _BM
    out = pl.pallas_call(
        _mcl_kernel,
        grid=(nsteps,),
        in_specs=[
            pl.BlockSpec((_BM, d), lambda i: (i, 0)),
            pl.BlockSpec(centers.shape, lambda i: (0, 0)),
            pl.BlockSpec((_BM, 1), lambda i: (i, 0)),
        ],
        out_specs=pl.BlockSpec(
            (1, 1), lambda i: (0, 0), memory_space=pltpu.SMEM
        ),
        out_shape=jax.ShapeDtypeStruct((1, 1), jnp.float32),
        scratch_shapes=[
            pltpu.VMEM((ncenters, d), jnp.float32),
            pltpu.VMEM((1, ncenters), jnp.float32),
            pltpu.SMEM((1, 1), jnp.float32),
            pltpu.SMEM((1, 1), jnp.float32),
        ],
    )(features, centers, labels2)
    return out[0, 0]


# speedup vs baseline: 1.2457x; 1.2457x over previous
import jax
import jax.numpy as jnp
from jax.experimental import pallas as pl
from jax.experimental.pallas import tpu as pltpu

_EPS = 1e-6
_D = 256
_BM = 4096


def _mcl_kernel(f_ref, c_ref, l_ref, out_ref, cs_ref, cn_ref, acc_s, acc_n):
    i = pl.program_id(0)
    nsteps = pl.num_programs(0)

    @pl.when(i == 0)
    def _init():
        acc_s[0, 0] = 0.0
        acc_n[0, 0] = 0.0
        c = c_ref[...]
        cs_ref[...] = -2.0 * c
        cn_ref[...] = (
            jnp.sum(c * c, axis=1) - (2.0 * _EPS) * jnp.sum(c, axis=1)
        )[None, :]

    f = f_ref[...]
    dot = jax.lax.dot_general(
        f, cs_ref[...], (((1,), (1,)), ((), ())),
        preferred_element_type=jnp.float32,
    )
    t = dot + cn_ref[...]
    m = jnp.min(t, axis=1, keepdims=True)
    rn = jnp.sum(f * f, axis=1, keepdims=True) + (2.0 * _EPS) * jnp.sum(
        f, axis=1, keepdims=True
    )
    min_d = jnp.sqrt(jnp.maximum(m + rn + (_D * _EPS * _EPS), 0.0))
    mask = (l_ref[...] == 0).astype(jnp.float32)
    acc_s[0, 0] += jnp.sum(mask * min_d)
    acc_n[0, 0] += jnp.sum(mask)

    @pl.when(i == nsteps - 1)
    def _fin():
        out_ref[0, 0] = acc_s[0, 0] / (acc_n[0, 0] + 1e-5)


def kernel(features, labels, centers):
    batch, d = features.shape
    ncenters = centers.shape[0]
    labels2 = labels.reshape(batch, 1)
    nsteps = batch // _BM
    out = pl.pallas_call(
        _mcl_kernel,
        grid=(nsteps,),
        in_specs=[
            pl.BlockSpec((_BM, d), lambda i: (i, 0)),
            pl.BlockSpec(centers.shape, lambda i: (0, 0)),
            pl.BlockSpec((_BM, 1), lambda i: (i, 0)),
        ],
        out_specs=pl.BlockSpec(
            (1, 1), lambda i: (0, 0), memory_space=pltpu.SMEM
        ),
        out_shape=jax.ShapeDtypeStruct((1, 1), jnp.float32),
        scratch_shapes=[
            pltpu.VMEM((ncenters, d), jnp.float32),
            pltpu.VMEM((1, ncenters), jnp.float32),
            pltpu.SMEM((1, 1), jnp.float32),
            pltpu.SMEM((1, 1), jnp.float32),
        ],
    )(features, centers, labels2)
    return out[0, 0]
